# baseline (speedup 1.0000x reference)
import jax
import jax.numpy as jnp
from jax import lax
from jax.experimental import pallas as pl
from jax.experimental.pallas import tpu as pltpu
from jax.experimental.pallas import tpu_sc as plsc

NUM_NODES = 100000
MEM_DIM = 144
OUT_DIM = 128
NC, NS = 2, 16
NW = NC * NS

BATCH = 16384
BLK = 8192
RS_BLK = 4096
B_PER_TILE = BATCH // NW
N_BLOCKS = BATCH // BLK
RS_BLOCKS = BATCH // RS_BLK
P_BLK = 12800
P_GRID = (NUM_NODES + P_BLK - 1) // P_BLK


def _rowsum_body(updt_ref, r_ref):
    r_ref[...] = jnp.sum(updt_ref[...], axis=0, keepdims=True).reshape(1, 1, RS_BLK)


def _rowsum(upd_t):
    r3 = pl.pallas_call(
        _rowsum_body,
        grid=(RS_BLOCKS,),
        in_specs=[pl.BlockSpec((MEM_DIM, RS_BLK), lambda i: (0, i))],
        out_specs=pl.BlockSpec((1, 1, RS_BLK), lambda i: (i, 0, 0)),
        out_shape=jax.ShapeDtypeStruct((RS_BLOCKS, 1, RS_BLK), jnp.float32),
    )(upd_t)
    return r3.reshape(BATCH)


def _proj_body(embt_ref, w_ref, p_ref):
    acc = jax.lax.dot_general(
        embt_ref[...], w_ref[...],
        dimension_numbers=(((0,), (0,)), ((), ())),
        preferred_element_type=jnp.float32,
    )
    p_ref[...] = pltpu.bitcast(acc.astype(jnp.bfloat16), jnp.int32)


def _project(emb_t, W_q):
    return pl.pallas_call(
        _proj_body,
        grid=(P_GRID,),
        in_specs=[
            pl.BlockSpec((MEM_DIM, P_BLK), lambda i: (0, i)),
            pl.BlockSpec((MEM_DIM, OUT_DIM), lambda i: (0, 0)),
        ],
        out_specs=pl.BlockSpec((P_BLK // 2, OUT_DIM), lambda i: (i, 0)),
        out_shape=jax.ShapeDtypeStruct((P_GRID * P_BLK // 2, OUT_DIM), jnp.int32),
    )(emb_t, W_q)


def _sc_s_body(ids_hbm, r_hbm, s_hbm, idx_all, r_all, idx_loc, s_loc, table):
    cid = lax.axis_index("c")
    sid = lax.axis_index("s")
    wid = cid * NS + sid
    base = wid * B_PER_TILE

    @pl.when(sid == 0)
    def _():
        pltpu.sync_copy(ids_hbm, idx_all)
        pltpu.sync_copy(r_hbm, r_all)
        pltpu.sync_copy(r_all, table.at[idx_all])

    plsc.subcore_barrier()

    pltpu.sync_copy(ids_hbm.at[pl.ds(base, B_PER_TILE)], idx_loc)
    pltpu.sync_copy(table.at[idx_loc], s_loc)
    pltpu.sync_copy(s_loc, s_hbm.at[pl.ds(base, B_PER_TILE)])


def _sc_resolve(ids, r1d):
    mesh = plsc.VectorSubcoreMesh(core_axis_name="c", subcore_axis_name="s")
    fn = pl.kernel(
        _sc_s_body,
        out_type=jax.ShapeDtypeStruct((BATCH,), jnp.float32),
        mesh=mesh,
        scratch_types=[
            pltpu.VMEM((BATCH,), jnp.int32),
            pltpu.VMEM((BATCH,), jnp.float32),
            pltpu.VMEM((B_PER_TILE,), jnp.int32),
            pltpu.VMEM((B_PER_TILE,), jnp.float32),
            pltpu.VMEM_SHARED((NUM_NODES,), jnp.float32),
        ],
    )
    return fn(ids, r1d)


def _unpack_rows(idx_loc, rows, half, off):
    def body(g, _):
        pv = jnp.bitwise_and(idx_loc[pl.ds(off + g * 16, 16)], 1)
        shv = jnp.left_shift(1 - pv, 4)
        for j in range(16):
            shamt = shv[j]
            for v in range(OUT_DIM // 16):
                x = rows[g * 16 + j, pl.ds(v * 16, 16)]
                x = jnp.bitwise_and(jnp.left_shift(x, shamt),
                                    jnp.int32(-65536))
                rows[g * 16 + j, pl.ds(v * 16, 16)] = x
        return 0

    lax.fori_loop(0, half // 16, body, 0)


def _sc_g_body(ids_hbm, p_hbm, g_hbm, idx_loc, idx2, rows0, rows1, sem0, sem1):
    cid = lax.axis_index("c")
    sid = lax.axis_index("s")
    wid = cid * NS + sid
    base = wid * B_PER_TILE
    half = B_PER_TILE // 2

    pltpu.sync_copy(ids_hbm.at[pl.ds(base, B_PER_TILE)], idx_loc)
    for v in range(B_PER_TILE // 16):
        idx2[pl.ds(v * 16, 16)] = lax.shift_right_logical(
            idx_loc[pl.ds(v * 16, 16)], 1)
    cp0 = pltpu.async_copy(p_hbm.at[idx2.at[pl.ds(0, half)]], rows0, sem0)
    cp1 = pltpu.async_copy(p_hbm.at[idx2.at[pl.ds(half, half)]], rows1, sem1)
    cp0.wait()
    _unpack_rows(idx_loc, rows0, half, 0)
    pltpu.sync_copy(rows0, g_hbm.at[pl.ds(base, half)])
    cp1.wait()
    _unpack_rows(idx_loc, rows1, half, half)
    pltpu.sync_copy(rows1, g_hbm.at[pl.ds(base + half, half)])


def _sc_gather(ids, p_table):
    mesh = plsc.VectorSubcoreMesh(core_axis_name="c", subcore_axis_name="s")
    fn = pl.kernel(
        _sc_g_body,
        out_type=jax.ShapeDtypeStruct((BATCH, OUT_DIM), jnp.int32),
        mesh=mesh,
        scratch_types=[
            pltpu.VMEM((B_PER_TILE,), jnp.int32),
            pltpu.VMEM((B_PER_TILE,), jnp.int32),
            pltpu.VMEM((B_PER_TILE // 2, OUT_DIM), jnp.int32),
            pltpu.VMEM((B_PER_TILE // 2, OUT_DIM), jnp.int32),
            pltpu.SemaphoreType.DMA,
            pltpu.SemaphoreType.DMA,
        ],
    )
    return fn(ids, p_table)


def _combine_body(s_ref, g_ref, w_ref, b_ref, o_ref):
    c = jnp.sum(w_ref[...], axis=0, keepdims=True)
    g = jax.lax.bitcast_convert_type(g_ref[...], jnp.float32)
    logits = g + s_ref[...] * c + b_ref[...]
    o_ref[...] = jax.nn.sigmoid(logits)


def _combine(s2, G, W_q, b2):
    return pl.pallas_call(
        _combine_body,
        grid=(N_BLOCKS,),
        in_specs=[
            pl.BlockSpec((BLK, 1), lambda i: (i, 0)),
            pl.BlockSpec((BLK, OUT_DIM), lambda i: (i, 0)),
            pl.BlockSpec((MEM_DIM, OUT_DIM), lambda i: (0, 0)),
            pl.BlockSpec((1, OUT_DIM), lambda i: (0, 0)),
        ],
        out_specs=pl.BlockSpec((BLK, OUT_DIM), lambda i: (i, 0)),
        out_shape=jax.ShapeDtypeStruct((BATCH, OUT_DIM), jnp.float32),
    )(s2, G, W_q, b2)


def kernel(node_ids, updated_node_memories, node_memories, emb_table, W_q, b_q):
    del node_memories
    ids = node_ids.astype(jnp.int32)
    r = _rowsum(updated_node_memories.T)
    s = _sc_resolve(ids, r)
    P = _project(emb_table.T, W_q)
    G = _sc_gather(ids, P)
    return _combine(s.reshape(BATCH, 1), G, W_q, b_q.reshape(1, OUT_DIM))


# speedup vs baseline: 1.0084x; 1.0084x over previous
import jax
import jax.numpy as jnp
from jax import lax
from jax.experimental import pallas as pl
from jax.experimental.pallas import tpu as pltpu
from jax.experimental.pallas import tpu_sc as plsc

NUM_NODES = 100000
MEM_DIM = 144
OUT_DIM = 128
NC, NS = 2, 16
NW = NC * NS

BATCH = 16384
BLK = 8192
RS_BLK = 8192
B_PER_TILE = BATCH // NW
N_BLOCKS = BATCH // BLK
RS_BLOCKS = BATCH // RS_BLK
P_BLK = 12800
P_GRID = (NUM_NODES + P_BLK - 1) // P_BLK


def _rowsum_body(updt_ref, r_ref):
    r_ref[...] = jnp.sum(updt_ref[...], axis=0, keepdims=True).reshape(1, 1, RS_BLK)


def _rowsum(upd_t):
    r3 = pl.pallas_call(
        _rowsum_body,
        grid=(RS_BLOCKS,),
        in_specs=[pl.BlockSpec((MEM_DIM, RS_BLK), lambda i: (0, i))],
        out_specs=pl.BlockSpec((1, 1, RS_BLK), lambda i: (i, 0, 0)),
        out_shape=jax.ShapeDtypeStruct((RS_BLOCKS, 1, RS_BLK), jnp.float32),
    )(upd_t)
    return r3.reshape(BATCH)


def _proj_body(embt_ref, w_ref, p_ref):
    acc = jax.lax.dot_general(
        embt_ref[...], w_ref[...],
        dimension_numbers=(((0,), (0,)), ((), ())),
        preferred_element_type=jnp.float32,
    )
    p_ref[...] = pltpu.bitcast(acc.astype(jnp.bfloat16), jnp.int32)


def _project(emb_t, W_q):
    return pl.pallas_call(
        _proj_body,
        grid=(P_GRID,),
        in_specs=[
            pl.BlockSpec((MEM_DIM, P_BLK), lambda i: (0, i)),
            pl.BlockSpec((MEM_DIM, OUT_DIM), lambda i: (0, 0)),
        ],
        out_specs=pl.BlockSpec((P_BLK // 2, OUT_DIM), lambda i: (i, 0)),
        out_shape=jax.ShapeDtypeStruct((P_GRID * P_BLK // 2, OUT_DIM), jnp.int32),
    )(emb_t, W_q)


def _sc_s_body(ids_hbm, r_hbm, s_hbm, idx_all, r_all, idx_loc, s_loc, table):
    cid = lax.axis_index("c")
    sid = lax.axis_index("s")
    wid = cid * NS + sid
    base = wid * B_PER_TILE

    @pl.when(sid == 0)
    def _():
        pltpu.sync_copy(ids_hbm, idx_all)
        pltpu.sync_copy(r_hbm, r_all)
        pltpu.sync_copy(r_all, table.at[idx_all])

    plsc.subcore_barrier()

    pltpu.sync_copy(ids_hbm.at[pl.ds(base, B_PER_TILE)], idx_loc)
    pltpu.sync_copy(table.at[idx_loc], s_loc)
    pltpu.sync_copy(s_loc, s_hbm.at[pl.ds(base, B_PER_TILE)])


def _sc_resolve(ids, r1d):
    mesh = plsc.VectorSubcoreMesh(core_axis_name="c", subcore_axis_name="s")
    fn = pl.kernel(
        _sc_s_body,
        out_type=jax.ShapeDtypeStruct((BATCH,), jnp.float32),
        mesh=mesh,
        scratch_types=[
            pltpu.VMEM((BATCH,), jnp.int32),
            pltpu.VMEM((BATCH,), jnp.float32),
            pltpu.VMEM((B_PER_TILE,), jnp.int32),
            pltpu.VMEM((B_PER_TILE,), jnp.float32),
            pltpu.VMEM_SHARED((NUM_NODES,), jnp.float32),
        ],
    )
    return fn(ids, r1d)


def _unpack_rows(idx_loc, rows, half, off):
    def body(g, _):
        pv = jnp.bitwise_and(idx_loc[pl.ds(off + g * 16, 16)], 1)
        shv = jnp.left_shift(1 - pv, 4)
        for j in range(16):
            shamt = shv[j]
            for v in range(OUT_DIM // 16):
                x = rows[g * 16 + j, pl.ds(v * 16, 16)]
                x = jnp.bitwise_and(jnp.left_shift(x, shamt),
                                    jnp.int32(-65536))
                rows[g * 16 + j, pl.ds(v * 16, 16)] = x
        return 0

    lax.fori_loop(0, half // 16, body, 0)


def _sc_g_body(ids_hbm, p_hbm, g_hbm, idx_loc, idx2, rows0, rows1, sem0, sem1):
    cid = lax.axis_index("c")
    sid = lax.axis_index("s")
    wid = cid * NS + sid
    base = wid * B_PER_TILE
    half = B_PER_TILE // 2

    pltpu.sync_copy(ids_hbm.at[pl.ds(base, B_PER_TILE)], idx_loc)
    for v in range(B_PER_TILE // 16):
        idx2[pl.ds(v * 16, 16)] = lax.shift_right_logical(
            idx_loc[pl.ds(v * 16, 16)], 1)
    cp0 = pltpu.async_copy(p_hbm.at[idx2.at[pl.ds(0, half)]], rows0, sem0)
    cp1 = pltpu.async_copy(p_hbm.at[idx2.at[pl.ds(half, half)]], rows1, sem1)
    cp0.wait()
    _unpack_rows(idx_loc, rows0, half, 0)
    pltpu.sync_copy(rows0, g_hbm.at[pl.ds(base, half)])
    cp1.wait()
    _unpack_rows(idx_loc, rows1, half, half)
    pltpu.sync_copy(rows1, g_hbm.at[pl.ds(base + half, half)])


def _sc_gather(ids, p_table):
    mesh = plsc.VectorSubcoreMesh(core_axis_name="c", subcore_axis_name="s")
    fn = pl.kernel(
        _sc_g_body,
        out_type=jax.ShapeDtypeStruct((BATCH, OUT_DIM), jnp.int32),
        mesh=mesh,
        scratch_types=[
            pltpu.VMEM((B_PER_TILE,), jnp.int32),
            pltpu.VMEM((B_PER_TILE,), jnp.int32),
            pltpu.VMEM((B_PER_TILE // 2, OUT_DIM), jnp.int32),
            pltpu.VMEM((B_PER_TILE // 2, OUT_DIM), jnp.int32),
            pltpu.SemaphoreType.DMA,
            pltpu.SemaphoreType.DMA,
        ],
    )
    return fn(ids, p_table)


def _combine_body(s_ref, g_ref, w_ref, b_ref, o_ref):
    c = jnp.sum(w_ref[...], axis=0, keepdims=True)
    g = jax.lax.bitcast_convert_type(g_ref[...], jnp.float32)
    logits = g + s_ref[...] * c + b_ref[...]
    o_ref[...] = jax.nn.sigmoid(logits)


def _combine(s2, G, W_q, b2):
    return pl.pallas_call(
        _combine_body,
        grid=(N_BLOCKS,),
        in_specs=[
            pl.BlockSpec((BLK, 1), lambda i: (i, 0)),
            pl.BlockSpec((BLK, OUT_DIM), lambda i: (i, 0)),
            pl.BlockSpec((MEM_DIM, OUT_DIM), lambda i: (0, 0)),
            pl.BlockSpec((1, OUT_DIM), lambda i: (0, 0)),
        ],
        out_specs=pl.BlockSpec((BLK, OUT_DIM), lambda i: (i, 0)),
        out_shape=jax.ShapeDtypeStruct((BATCH, OUT_DIM), jnp.float32),
    )(s2, G, W_q, b2)


def kernel(node_ids, updated_node_memories, node_memories, emb_table, W_q, b_q):
    del node_memories
    ids = node_ids.astype(jnp.int32)
    r = _rowsum(updated_node_memories.T)
    s = _sc_resolve(ids, r)
    P = _project(emb_table.T, W_q)
    G = _sc_gather(ids, P)
    return _combine(s.reshape(BATCH, 1), G, W_q, b_q.reshape(1, OUT_DIM))


# speedup vs baseline: 1.0228x; 1.0142x over previous
import jax
import jax.numpy as jnp
from jax import lax
from jax.experimental import pallas as pl
from jax.experimental.pallas import tpu as pltpu
from jax.experimental.pallas import tpu_sc as plsc

NUM_NODES = 100000
MEM_DIM = 144
OUT_DIM = 128
NC, NS = 2, 16
NW = NC * NS

BATCH = 16384
BLK = 8192
RS_BLK = 8192
B_PER_TILE = BATCH // NW
N_BLOCKS = BATCH // BLK
RS_BLOCKS = BATCH // RS_BLK
P_BLK = 12800
P_GRID = (NUM_NODES + P_BLK - 1) // P_BLK


def _rowsum_body(updt_ref, r_ref):
    r_ref[...] = jnp.sum(updt_ref[...], axis=0, keepdims=True).reshape(1, 1, RS_BLK)


def _rowsum(upd_t):
    r3 = pl.pallas_call(
        _rowsum_body,
        grid=(RS_BLOCKS,),
        in_specs=[pl.BlockSpec((MEM_DIM, RS_BLK), lambda i: (0, i))],
        out_specs=pl.BlockSpec((1, 1, RS_BLK), lambda i: (i, 0, 0)),
        out_shape=jax.ShapeDtypeStruct((RS_BLOCKS, 1, RS_BLK), jnp.float32),
    )(upd_t)
    return r3.reshape(BATCH)


def _proj_body(embt_ref, w_ref, p_ref):
    acc = jax.lax.dot_general(
        embt_ref[...], w_ref[...],
        dimension_numbers=(((0,), (0,)), ((), ())),
        preferred_element_type=jnp.float32,
    )
    p_ref[...] = pltpu.bitcast(acc.astype(jnp.bfloat16), jnp.int32)


def _project(emb_t, W_q):
    return pl.pallas_call(
        _proj_body,
        grid=(P_GRID,),
        in_specs=[
            pl.BlockSpec((MEM_DIM, P_BLK), lambda i: (0, i)),
            pl.BlockSpec((MEM_DIM, OUT_DIM), lambda i: (0, 0)),
        ],
        out_specs=pl.BlockSpec((P_BLK // 2, OUT_DIM), lambda i: (i, 0)),
        out_shape=jax.ShapeDtypeStruct((P_GRID * P_BLK // 2, OUT_DIM), jnp.int32),
    )(emb_t, W_q)


def _sc_s_body(ids_hbm, r_hbm, s_hbm, idx_all, r_all, idx_loc, s_loc, table):
    cid = lax.axis_index("c")
    sid = lax.axis_index("s")
    wid = cid * NS + sid
    base = wid * B_PER_TILE

    @pl.when(sid == 0)
    def _():
        pltpu.sync_copy(ids_hbm, idx_all)
        pltpu.sync_copy(r_hbm, r_all)
        pltpu.sync_copy(r_all, table.at[idx_all])

    plsc.subcore_barrier()

    pltpu.sync_copy(ids_hbm.at[pl.ds(base, B_PER_TILE)], idx_loc)
    pltpu.sync_copy(table.at[idx_loc], s_loc)
    pltpu.sync_copy(s_loc, s_hbm.at[pl.ds(base, B_PER_TILE)])


def _sc_resolve(ids, r1d):
    mesh = plsc.VectorSubcoreMesh(core_axis_name="c", subcore_axis_name="s")
    fn = pl.kernel(
        _sc_s_body,
        out_type=jax.ShapeDtypeStruct((BATCH,), jnp.float32),
        mesh=mesh,
        scratch_types=[
            pltpu.VMEM((BATCH,), jnp.int32),
            pltpu.VMEM((BATCH,), jnp.float32),
            pltpu.VMEM((B_PER_TILE,), jnp.int32),
            pltpu.VMEM((B_PER_TILE,), jnp.float32),
            pltpu.VMEM_SHARED((NUM_NODES,), jnp.float32),
        ],
    )
    return fn(ids, r1d)


def _unpack_rows(idx_loc, rows, half, off):
    def body(g, _):
        pv = jnp.bitwise_and(idx_loc[pl.ds(off + g * 16, 16)], 1)
        shv = jnp.left_shift(1 - pv, 4)
        for j in range(16):
            shamt = shv[j]
            for v in range(OUT_DIM // 16):
                x = rows[g * 16 + j, pl.ds(v * 16, 16)]
                x = jnp.bitwise_and(jnp.left_shift(x, shamt),
                                    jnp.int32(-65536))
                rows[g * 16 + j, pl.ds(v * 16, 16)] = x
        return 0

    lax.fori_loop(0, half // 16, body, 0)


def _sc_g_body(ids_hbm, p_hbm, g_hbm, idx_loc, idx2, rows0, rows1, sem0, sem1):
    cid = lax.axis_index("c")
    sid = lax.axis_index("s")
    wid = cid * NS + sid
    base = wid * B_PER_TILE
    half = B_PER_TILE // 2

    pltpu.sync_copy(ids_hbm.at[pl.ds(base, B_PER_TILE)], idx_loc)
    for v in range(B_PER_TILE // 16):
        idx2[pl.ds(v * 16, 16)] = lax.shift_right_logical(
            idx_loc[pl.ds(v * 16, 16)], 1)
    cp0 = pltpu.async_copy(p_hbm.at[idx2.at[pl.ds(0, half)]], rows0, sem0)
    cp1 = pltpu.async_copy(p_hbm.at[idx2.at[pl.ds(half, half)]], rows1, sem1)
    cp0.wait()
    _unpack_rows(idx_loc, rows0, half, 0)
    w0 = pltpu.async_copy(rows0, g_hbm.at[pl.ds(base, half)], sem0)
    cp1.wait()
    _unpack_rows(idx_loc, rows1, half, half)
    w0.wait()
    pltpu.sync_copy(rows1, g_hbm.at[pl.ds(base + half, half)])


def _sc_gather(ids, p_table):
    mesh = plsc.VectorSubcoreMesh(core_axis_name="c", subcore_axis_name="s")
    fn = pl.kernel(
        _sc_g_body,
        out_type=jax.ShapeDtypeStruct((BATCH, OUT_DIM), jnp.int32),
        mesh=mesh,
        scratch_types=[
            pltpu.VMEM((B_PER_TILE,), jnp.int32),
            pltpu.VMEM((B_PER_TILE,), jnp.int32),
            pltpu.VMEM((B_PER_TILE // 2, OUT_DIM), jnp.int32),
            pltpu.VMEM((B_PER_TILE // 2, OUT_DIM), jnp.int32),
            pltpu.SemaphoreType.DMA,
            pltpu.SemaphoreType.DMA,
        ],
    )
    return fn(ids, p_table)


def _combine_body(s_ref, g_ref, w_ref, b_ref, o_ref):
    c = jnp.sum(w_ref[...], axis=0, keepdims=True)
    g = jax.lax.bitcast_convert_type(g_ref[...], jnp.float32)
    logits = g + s_ref[...] * c + b_ref[...]
    o_ref[...] = jax.nn.sigmoid(logits)


def _combine(s2, G, W_q, b2):
    return pl.pallas_call(
        _combine_body,
        grid=(N_BLOCKS,),
        in_specs=[
            pl.BlockSpec((BLK, 1), lambda i: (i, 0)),
            pl.BlockSpec((BLK, OUT_DIM), lambda i: (i, 0)),
            pl.BlockSpec((MEM_DIM, OUT_DIM), lambda i: (0, 0)),
            pl.BlockSpec((1, OUT_DIM), lambda i: (0, 0)),
        ],
        out_specs=pl.BlockSpec((BLK, OUT_DIM), lambda i: (i, 0)),
        out_shape=jax.ShapeDtypeStruct((BATCH, OUT_DIM), jnp.float32),
    )(s2, G, W_q, b2)


def kernel(node_ids, updated_node_memories, node_memories, emb_table, W_q, b_q):
    del node_memories
    ids = node_ids.astype(jnp.int32)
    r = _rowsum(updated_node_memories.T)
    s = _sc_resolve(ids, r)
    P = _project(emb_table.T, W_q)
    G = _sc_gather(ids, P)
    return _combine(s.reshape(BATCH, 1), G, W_q, b_q.reshape(1, OUT_DIM))
